# baseline (speedup 1.0000x reference)
import functools

import jax
import jax.numpy as jnp
from jax import lax
from jax.experimental import pallas as pl
from jax.experimental.pallas import tpu as pltpu
from jax.experimental.pallas import tpu_sc as plsc

N = 10000
E = 320000
NUM_GRAPHS = 256
CHUNK = 128
K_BLK = 32
EI_PAD = 6400
N_TILE = 624
N_TAIL = N - 16 * N_TILE
N_CHUNKS = E // CHUNK

_PREC = jax.lax.Precision.HIGHEST


def _dot(a, b, dim):
    return jax.lax.dot_general(
        a, b, (((1,), (dim,)), ((), ())),
        precision=_PREC, preferred_element_type=jnp.float32)


def _sc_agg(h, src, dst, zeros, feat_split):
    mesh = plsc.VectorSubcoreMesh(core_axis_name="c", subcore_axis_name="s")
    n_workers = 16 if feat_split else 32
    base_cnt = N_CHUNKS // n_workers
    n_rem = N_CHUNKS % n_workers
    max_cnt = base_cnt + (1 if n_rem else 0)
    n_blocks = -(-max_cnt // K_BLK)
    n_idx = K_BLK * CHUNK

    @functools.partial(
        pl.kernel,
        mesh=mesh,
        out_type=jax.ShapeDtypeStruct((2 * N, 128), jnp.float32),
        scratch_types=[
            pltpu.VMEM((n_idx,), jnp.int32),
            pltpu.VMEM((n_idx,), jnp.int32),
            pltpu.VMEM((n_idx,), jnp.int32),
            pltpu.VMEM((n_idx,), jnp.int32),
            pltpu.VMEM((CHUNK,), jnp.int32),
            pltpu.VMEM((CHUNK,), jnp.int32),
            pltpu.VMEM((CHUNK, 128), jnp.float32),
            pltpu.VMEM((CHUNK, 128), jnp.float32),
            pltpu.VMEM_SHARED((N, 128), jnp.float32),
            pltpu.SemaphoreType.DMA,
            pltpu.SemaphoreType.DMA,
            pltpu.SemaphoreType.DMA,
            pltpu.SemaphoreType.DMA,
            pltpu.SemaphoreType.DMA,
            pltpu.SemaphoreType.DMA,
        ],
    )
    def k(h_hbm, src_hbm, dst_hbm, z_hbm, out_hbm, src_a, src_b, dst_a,
          dst_b, dst_v0, dst_v1, rows0, rows1, acc_sh, sem0, sem1,
          ssem0, ssem1, isem0, isem1):
        c = lax.axis_index("c")
        s = lax.axis_index("s")
        w = s if feat_split else c * 16 + s
        start = w * base_cnt + jnp.minimum(w, n_rem)
        cnt = base_cnt + (w < n_rem).astype(jnp.int32)

        dst_bufs = (dst_v0, dst_v1)
        row_bufs = (rows0, rows1)
        src_blks = (src_a, src_b)
        dst_blks = (dst_a, dst_b)
        sems = (sem0, sem1)
        ssems = (ssem0, ssem1)
        isems = (isem0, isem1)
        row_off = jnp.full((16,), c * N, dtype=jnp.int32)

        pltpu.sync_copy(z_hbm, acc_sh.at[pl.ds(s * N_TILE, N_TILE)])

        @pl.when(s == 0)
        def _():
            pltpu.sync_copy(z_hbm.at[pl.ds(0, N_TAIL)],
                            acc_sh.at[pl.ds(16 * N_TILE, N_TAIL)])

        plsc.subcore_barrier()

        def issue_gather(src_all, i, b, guard):
            @pl.when(guard)
            def _():
                pltpu.async_copy(
                    h_hbm.at[src_all.at[pl.ds(i * CHUNK, CHUNK)]],
                    row_bufs[b], sems[b])

        def drain_scatter(b, guard):
            @pl.when(guard)
            def _():
                pltpu.make_async_copy(row_bufs[b], acc_sh.at[dst_bufs[b]],
                                      ssems[b]).wait()

        def issue_idx_load(blk, p):
            @pl.when(blk < n_blocks)
            def _():
                ib = (start + blk * K_BLK) * CHUNK
                pltpu.async_copy(src_hbm.at[pl.ds(ib, n_idx)],
                                 src_blks[p], isems[p])
                pltpu.async_copy(dst_hbm.at[pl.ds(ib, n_idx)],
                                 dst_blks[p], isems[p])

        def wait_idx_load(blk, p):
            @pl.when(blk < n_blocks)
            def _():
                pltpu.make_async_copy(src_hbm.at[pl.ds(0, n_idx)],
                                      src_blks[p], isems[p]).wait()
                pltpu.make_async_copy(dst_hbm.at[pl.ds(0, n_idx)],
                                      dst_blks[p], isems[p]).wait()

        def block_body(blk, p):
            src_all = src_blks[p]
            dst_all = dst_blks[p]
            kk0 = blk * K_BLK
            wait_idx_load(blk, p)
            issue_idx_load(blk + 1, 1 - p)

            if feat_split:
                def adj(i, cy):
                    sl = pl.ds(i * 16, 16)
                    src_all[sl] = src_all[sl] + row_off
                    return cy

                lax.fori_loop(0, n_idx // 16, adj, 0)

            drain_scatter(0, (kk0 >= 2) & (kk0 - 2 < cnt))
            issue_gather(src_all, 0, 0, kk0 < cnt)

            def step(i, b, prefetch):
                kk = kk0 + i

                if prefetch:
                    drain_scatter(1 - b, (kk >= 1) & (kk - 1 < cnt))
                    issue_gather(src_all, i + 1, 1 - b, kk + 1 < cnt)

                @pl.when(kk < cnt)
                def _():
                    for t in range(CHUNK // 16):
                        dst_bufs[b][pl.ds(t * 16, 16)] = (
                            dst_all[pl.ds(i * CHUNK + t * 16, 16)])
                    pltpu.make_async_copy(
                        h_hbm.at[src_all.at[pl.ds(i * CHUNK, CHUNK)]],
                        row_bufs[b], sems[b]).wait()
                    pltpu.async_copy(row_bufs[b], acc_sh.at[dst_bufs[b]],
                                     ssems[b], add=True)

            def pair(jj, cy):
                step(jj * 2, 0, True)
                step(jj * 2 + 1, 1, True)
                return cy

            lax.fori_loop(0, K_BLK // 2 - 1, pair, 0)
            step(K_BLK - 2, 0, True)
            step(K_BLK - 1, 1, False)

        issue_idx_load(0, 0)

        def super_body(jj, carry):
            block_body(jj * 2, 0)
            block_body(jj * 2 + 1, 1)
            return carry

        n_super = -(-n_blocks // 2)
        lax.fori_loop(0, n_super, super_body, 0)

        kk_end = 2 * n_super * K_BLK
        drain_scatter(0, kk_end - 2 < cnt)
        drain_scatter(1, kk_end - 1 < cnt)
        plsc.subcore_barrier()

        pltpu.sync_copy(acc_sh.at[pl.ds(s * N_TILE, N_TILE)],
                        out_hbm.at[pl.ds(c * N + s * N_TILE, N_TILE)])

        @pl.when(s == 0)
        def _():
            pltpu.sync_copy(acc_sh.at[pl.ds(16 * N_TILE, N_TAIL)],
                            out_hbm.at[pl.ds(c * N + 16 * N_TILE, N_TAIL)])

    return k(h, src, dst, zeros)


BLK = 2000


def _tc_layer1(agg, x, w_rel, w_root, b):
    def body(a_ref, x_ref, wrel_ref, wroot_ref, b_ref, o_ref):
        wrel = wrel_ref[...]
        h = (_dot(a_ref[0], wrel, 1) + _dot(a_ref[1], wrel, 1)
             + _dot(x_ref[...], wroot_ref[...], 1) + b_ref[...])
        o_ref[...] = jnp.maximum(h, 0.0)

    return pl.pallas_call(
        body,
        grid=(N // BLK,),
        in_specs=[
            pl.BlockSpec((2, BLK, 128), lambda i: (0, i, 0)),
            pl.BlockSpec((BLK, 128), lambda i: (i, 0)),
            pl.BlockSpec((128, 128), lambda i: (0, 0)),
            pl.BlockSpec((128, 128), lambda i: (0, 0)),
            pl.BlockSpec((1, 128), lambda i: (0, 0)),
        ],
        out_specs=pl.BlockSpec((BLK, 128), lambda i: (i, 0)),
        out_shape=jax.ShapeDtypeStruct((N, 128), jnp.float32),
    )(agg, x, w_rel, w_root, b.reshape(1, 128))


def _tc_layer2(agg, h1, w_rel, w_root, b):
    def body(a_ref, h_ref, wrel_ref, wroot_ref, b_ref, o_ref):
        wrel = wrel_ref[...]
        h = (_dot(a_ref[0], wrel, 1) + _dot(a_ref[1], wrel, 1)
             + _dot(h_ref[...], wroot_ref[...], 1) + b_ref[...])
        h = jnp.maximum(h, 0.0)
        o_ref[0] = h[:, :128]
        o_ref[1] = h[:, 128:]

    return pl.pallas_call(
        body,
        grid=(N // BLK,),
        in_specs=[
            pl.BlockSpec((2, BLK, 128), lambda i: (0, i, 0)),
            pl.BlockSpec((BLK, 128), lambda i: (i, 0)),
            pl.BlockSpec((256, 128), lambda i: (0, 0)),
            pl.BlockSpec((256, 128), lambda i: (0, 0)),
            pl.BlockSpec((1, 256), lambda i: (0, 0)),
        ],
        out_specs=pl.BlockSpec((2, BLK, 128), lambda i: (0, i, 0)),
        out_shape=jax.ShapeDtypeStruct((2, N, 128), jnp.float32),
    )(agg, h1, w_rel, w_root, b.reshape(1, 256))


def _tc_layer3_fc_pool(agg, h2, w_rel, w_root, b, wfc, bfc, gamma, beta,
                       batch3d):
    def body(a_ref, h_ref, wrel_ref, wroot_ref, b_ref, wfc_ref, bfc_ref,
             g_ref, be_ref, ids_ref, atom_ref, pool_ref):
        h3 = (_dot(a_ref[0], wrel_ref[...][:, :128], 1)
              + _dot(a_ref[1], wrel_ref[...][:, 128:], 1)
              + _dot(h_ref[0], wroot_ref[...][:, :128], 1)
              + _dot(h_ref[1], wroot_ref[...][:, 128:], 1)
              + b_ref[...])
        h3 = jnp.maximum(h3, 0.0)
        hf = _dot(h3, wfc_ref[...], 1) + bfc_ref[...]
        mu = jnp.mean(hf, axis=-1, keepdims=True)
        var = jnp.mean((hf - mu) ** 2, axis=-1, keepdims=True)
        ln = (hf - mu) * jax.lax.rsqrt(var + 1e-5) * g_ref[...] + be_ref[...]
        atom = jnp.maximum(ln, 0.0)
        atom_ref[...] = atom

        ids = ids_ref[0, 0, :]
        gids = jax.lax.broadcasted_iota(jnp.int32, (NUM_GRAPHS, BLK), 0)
        onehot = (gids == ids[None, :]).astype(jnp.float32)
        part = jax.lax.dot_general(
            onehot, atom, (((1,), (0,)), ((), ())),
            precision=_PREC, preferred_element_type=jnp.float32)

        @pl.when(pl.program_id(0) == 0)
        def _():
            pool_ref[...] = jnp.zeros_like(pool_ref)

        pool_ref[...] += part

    return pl.pallas_call(
        body,
        grid=(N // BLK,),
        in_specs=[
            pl.BlockSpec((2, BLK, 128), lambda i: (0, i, 0)),
            pl.BlockSpec((2, BLK, 128), lambda i: (0, i, 0)),
            pl.BlockSpec((256, 256), lambda i: (0, 0)),
            pl.BlockSpec((256, 256), lambda i: (0, 0)),
            pl.BlockSpec((1, 256), lambda i: (0, 0)),
            pl.BlockSpec((128, 256), lambda i: (0, 0)),
            pl.BlockSpec((1, 128), lambda i: (0, 0)),
            pl.BlockSpec((1, 128), lambda i: (0, 0)),
            pl.BlockSpec((1, 128), lambda i: (0, 0)),
            pl.BlockSpec((1, 1, BLK), lambda i: (i, 0, 0)),
        ],
        out_specs=[
            pl.BlockSpec((BLK, 128), lambda i: (i, 0)),
            pl.BlockSpec((NUM_GRAPHS, 128), lambda i: (0, 0)),
        ],
        out_shape=[
            jax.ShapeDtypeStruct((N, 128), jnp.float32),
            jax.ShapeDtypeStruct((NUM_GRAPHS, 128), jnp.float32),
        ],
    )(agg, h2, w_rel, w_root, b.reshape(1, 256), wfc, bfc.reshape(1, 128),
      gamma.reshape(1, 128), beta.reshape(1, 128), batch3d)


def kernel(x_l, edge_index_l, batch_l, W1_root, W1_rel, b1, W2_root, W2_rel,
           b2, W3_root, W3_rel, b3, Wfc, bfc, gamma, beta):
    pad = jnp.zeros((2, EI_PAD), jnp.int32)
    ei = jnp.concatenate([edge_index_l, pad], axis=1)
    src = ei[0]
    dst = ei[1]
    zeros = jnp.zeros((N_TILE, 128), jnp.float32)
    batch3d = batch_l.reshape(N // BLK, 1, BLK)

    agg1 = _sc_agg(x_l, src, dst, zeros, False).reshape(2, N, 128)
    h1 = _tc_layer1(agg1, x_l, W1_rel, W1_root, b1)

    agg2 = _sc_agg(h1, src, dst, zeros, False).reshape(2, N, 128)
    h2 = _tc_layer2(agg2, h1, W2_rel, W2_root, b2)

    agg3 = _sc_agg(h2.reshape(2 * N, 128), src, dst, zeros,
                   True).reshape(2, N, 128)
    x_atom, x_pool = _tc_layer3_fc_pool(agg3, h2, W3_rel, W3_root, b3, Wfc,
                                        bfc, gamma, beta, batch3d)
    return (x_atom, x_pool)


# speedup vs baseline: 1.0550x; 1.0550x over previous
import functools

import jax
import jax.numpy as jnp
from jax import lax
from jax.experimental import pallas as pl
from jax.experimental.pallas import tpu as pltpu
from jax.experimental.pallas import tpu_sc as plsc

N = 10000
E = 320000
NUM_GRAPHS = 256
CHUNK = 128
K_BLK = 48
EI_PAD = 4608
N_TILE = 624
N_TAIL = N - 16 * N_TILE
N_CHUNKS = E // CHUNK

_PREC = jax.lax.Precision.HIGHEST


def _dot(a, b, dim):
    return jax.lax.dot_general(
        a, b, (((1,), (dim,)), ((), ())),
        precision=_PREC, preferred_element_type=jnp.float32)


def _sc_agg(h, src, dst, zeros, feat_split):
    mesh = plsc.VectorSubcoreMesh(core_axis_name="c", subcore_axis_name="s")
    n_workers = 16 if feat_split else 32
    base_cnt = N_CHUNKS // n_workers
    n_rem = N_CHUNKS % n_workers
    max_cnt = base_cnt + (1 if n_rem else 0)
    n_blocks = -(-max_cnt // K_BLK)
    n_idx = K_BLK * CHUNK

    @functools.partial(
        pl.kernel,
        mesh=mesh,
        out_type=jax.ShapeDtypeStruct((2 * N, 128), jnp.float32),
        scratch_types=[
            pltpu.VMEM((n_idx,), jnp.int32),
            pltpu.VMEM((n_idx,), jnp.int32),
            pltpu.VMEM((CHUNK,), jnp.int32),
            pltpu.VMEM((CHUNK,), jnp.int32),
            pltpu.VMEM((CHUNK, 128), jnp.float32),
            pltpu.VMEM((CHUNK, 128), jnp.float32),
            pltpu.VMEM_SHARED((N, 128), jnp.float32),
            pltpu.SemaphoreType.DMA,
            pltpu.SemaphoreType.DMA,
            pltpu.SemaphoreType.DMA,
            pltpu.SemaphoreType.DMA,
        ],
    )
    def k(h_hbm, src_hbm, dst_hbm, z_hbm, out_hbm, src_all, dst_all,
          dst_v0, dst_v1, rows0, rows1, acc_sh, sem0, sem1, ssem0, ssem1):
        c = lax.axis_index("c")
        s = lax.axis_index("s")
        w = s if feat_split else c * 16 + s
        start = w * base_cnt + jnp.minimum(w, n_rem)
        cnt = base_cnt + (w < n_rem).astype(jnp.int32)

        dst_bufs = (dst_v0, dst_v1)
        row_bufs = (rows0, rows1)
        sems = (sem0, sem1)
        ssems = (ssem0, ssem1)
        row_off = jnp.full((16,), c * N, dtype=jnp.int32)

        pltpu.sync_copy(z_hbm, acc_sh.at[pl.ds(s * N_TILE, N_TILE)])

        @pl.when(s == 0)
        def _():
            pltpu.sync_copy(z_hbm.at[pl.ds(0, N_TAIL)],
                            acc_sh.at[pl.ds(16 * N_TILE, N_TAIL)])

        plsc.subcore_barrier()

        def issue_gather(i, b, guard):
            @pl.when(guard)
            def _():
                pltpu.async_copy(
                    h_hbm.at[src_all.at[pl.ds(i * CHUNK, CHUNK)]],
                    row_bufs[b], sems[b])

        def drain_scatter(b, guard):
            @pl.when(guard)
            def _():
                pltpu.make_async_copy(row_bufs[b], acc_sh.at[dst_bufs[b]],
                                      ssems[b]).wait()

        def block_body(blk, carry):
            kk0 = blk * K_BLK
            ib = (start + kk0) * CHUNK
            pltpu.sync_copy(src_hbm.at[pl.ds(ib, n_idx)], src_all)
            pltpu.sync_copy(dst_hbm.at[pl.ds(ib, n_idx)], dst_all)

            if feat_split:
                def adj(i, cy):
                    sl = pl.ds(i * 16, 16)
                    src_all[sl] = src_all[sl] + row_off
                    return cy

                lax.fori_loop(0, n_idx // 16, adj, 0)

            drain_scatter(0, (kk0 >= 2) & (kk0 - 2 < cnt))
            issue_gather(0, 0, kk0 < cnt)

            def step(i, b, prefetch):
                kk = kk0 + i

                if prefetch:
                    drain_scatter(1 - b, (kk >= 1) & (kk - 1 < cnt))
                    issue_gather(i + 1, 1 - b, kk + 1 < cnt)

                @pl.when(kk < cnt)
                def _():
                    for t in range(CHUNK // 16):
                        dst_bufs[b][pl.ds(t * 16, 16)] = (
                            dst_all[pl.ds(i * CHUNK + t * 16, 16)])
                    pltpu.make_async_copy(
                        h_hbm.at[src_all.at[pl.ds(i * CHUNK, CHUNK)]],
                        row_bufs[b], sems[b]).wait()
                    pltpu.async_copy(row_bufs[b], acc_sh.at[dst_bufs[b]],
                                     ssems[b], add=True)

            def pair(jj, cy):
                step(jj * 2, 0, True)
                step(jj * 2 + 1, 1, True)
                return cy

            lax.fori_loop(0, K_BLK // 2 - 1, pair, 0)
            step(K_BLK - 2, 0, True)
            step(K_BLK - 1, 1, False)
            return carry

        lax.fori_loop(0, n_blocks, block_body, 0)

        kk_last = (n_blocks - 1) * K_BLK
        drain_scatter(0, (kk_last + K_BLK - 2 >= 0) & (kk_last + K_BLK - 2 < cnt))
        drain_scatter(1, kk_last + K_BLK - 1 < cnt)
        plsc.subcore_barrier()

        pltpu.sync_copy(acc_sh.at[pl.ds(s * N_TILE, N_TILE)],
                        out_hbm.at[pl.ds(c * N + s * N_TILE, N_TILE)])

        @pl.when(s == 0)
        def _():
            pltpu.sync_copy(acc_sh.at[pl.ds(16 * N_TILE, N_TAIL)],
                            out_hbm.at[pl.ds(c * N + 16 * N_TILE, N_TAIL)])

    return k(h, src, dst, zeros)


BLK = 2000


def _tc_root(x, w_root, b, w_out):
    halves = x.ndim == 3

    def body(x_ref, wroot_ref, b_ref, o_ref):
        if halves:
            r = (_dot(x_ref[0], wroot_ref[...][:, :128], 1)
                 + _dot(x_ref[1], wroot_ref[...][:, 128:], 1))
        else:
            r = _dot(x_ref[...], wroot_ref[...], 1)
        o_ref[...] = r + b_ref[...]

    x_spec = (pl.BlockSpec((2, BLK, 128), lambda i: (0, i, 0)) if halves
              else pl.BlockSpec((BLK, 128), lambda i: (i, 0)))
    return pl.pallas_call(
        body,
        grid=(N // BLK,),
        in_specs=[
            x_spec,
            pl.BlockSpec(w_root.shape, lambda i: (0, 0)),
            pl.BlockSpec((1, w_out), lambda i: (0, 0)),
        ],
        out_specs=pl.BlockSpec((BLK, w_out), lambda i: (i, 0)),
        out_shape=jax.ShapeDtypeStruct((N, w_out), jnp.float32),
    )(x, w_root, b.reshape(1, w_out))


def _tc_layer1(agg, r, w_rel):
    def body(a_ref, r_ref, wrel_ref, o_ref):
        wrel = wrel_ref[...]
        h = _dot(a_ref[0], wrel, 1) + _dot(a_ref[1], wrel, 1) + r_ref[...]
        o_ref[...] = jnp.maximum(h, 0.0)

    return pl.pallas_call(
        body,
        grid=(N // BLK,),
        in_specs=[
            pl.BlockSpec((2, BLK, 128), lambda i: (0, i, 0)),
            pl.BlockSpec((BLK, 128), lambda i: (i, 0)),
            pl.BlockSpec((128, 128), lambda i: (0, 0)),
        ],
        out_specs=pl.BlockSpec((BLK, 128), lambda i: (i, 0)),
        out_shape=jax.ShapeDtypeStruct((N, 128), jnp.float32),
    )(agg, r, w_rel)


def _tc_layer2(agg, r, w_rel):
    def body(a_ref, r_ref, wrel_ref, o_ref):
        wrel = wrel_ref[...]
        h = _dot(a_ref[0], wrel, 1) + _dot(a_ref[1], wrel, 1) + r_ref[...]
        h = jnp.maximum(h, 0.0)
        o_ref[0] = h[:, :128]
        o_ref[1] = h[:, 128:]

    return pl.pallas_call(
        body,
        grid=(N // BLK,),
        in_specs=[
            pl.BlockSpec((2, BLK, 128), lambda i: (0, i, 0)),
            pl.BlockSpec((BLK, 256), lambda i: (i, 0)),
            pl.BlockSpec((256, 128), lambda i: (0, 0)),
        ],
        out_specs=pl.BlockSpec((2, BLK, 128), lambda i: (0, i, 0)),
        out_shape=jax.ShapeDtypeStruct((2, N, 128), jnp.float32),
    )(agg, r, w_rel)


def _tc_layer3_fc_pool(agg, r, w_rel, wfc, bfc, gamma, beta, batch3d):
    def body(a_ref, r_ref, wrel_ref, wfc_ref, bfc_ref,
             g_ref, be_ref, ids_ref, atom_ref, pool_ref):
        h3 = (_dot(a_ref[0], wrel_ref[...][:, :128], 1)
              + _dot(a_ref[1], wrel_ref[...][:, 128:], 1)
              + r_ref[...])
        h3 = jnp.maximum(h3, 0.0)
        hf = _dot(h3, wfc_ref[...], 1) + bfc_ref[...]
        mu = jnp.mean(hf, axis=-1, keepdims=True)
        var = jnp.mean((hf - mu) ** 2, axis=-1, keepdims=True)
        ln = (hf - mu) * jax.lax.rsqrt(var + 1e-5) * g_ref[...] + be_ref[...]
        atom = jnp.maximum(ln, 0.0)
        atom_ref[...] = atom

        ids = ids_ref[0, 0, :]
        gids = jax.lax.broadcasted_iota(jnp.int32, (NUM_GRAPHS, BLK), 0)
        onehot = (gids == ids[None, :]).astype(jnp.float32)
        part = jax.lax.dot_general(
            onehot, atom, (((1,), (0,)), ((), ())),
            precision=_PREC, preferred_element_type=jnp.float32)

        @pl.when(pl.program_id(0) == 0)
        def _():
            pool_ref[...] = jnp.zeros_like(pool_ref)

        pool_ref[...] += part

    return pl.pallas_call(
        body,
        grid=(N // BLK,),
        in_specs=[
            pl.BlockSpec((2, BLK, 128), lambda i: (0, i, 0)),
            pl.BlockSpec((BLK, 256), lambda i: (i, 0)),
            pl.BlockSpec((256, 256), lambda i: (0, 0)),
            pl.BlockSpec((128, 256), lambda i: (0, 0)),
            pl.BlockSpec((1, 128), lambda i: (0, 0)),
            pl.BlockSpec((1, 128), lambda i: (0, 0)),
            pl.BlockSpec((1, 128), lambda i: (0, 0)),
            pl.BlockSpec((1, 1, BLK), lambda i: (i, 0, 0)),
        ],
        out_specs=[
            pl.BlockSpec((BLK, 128), lambda i: (i, 0)),
            pl.BlockSpec((NUM_GRAPHS, 128), lambda i: (0, 0)),
        ],
        out_shape=[
            jax.ShapeDtypeStruct((N, 128), jnp.float32),
            jax.ShapeDtypeStruct((NUM_GRAPHS, 128), jnp.float32),
        ],
    )(agg, r, w_rel, wfc, bfc.reshape(1, 128),
      gamma.reshape(1, 128), beta.reshape(1, 128), batch3d)


def kernel(x_l, edge_index_l, batch_l, W1_root, W1_rel, b1, W2_root, W2_rel,
           b2, W3_root, W3_rel, b3, Wfc, bfc, gamma, beta):
    pad = jnp.zeros((2, EI_PAD), jnp.int32)
    ei = jnp.concatenate([edge_index_l, pad], axis=1)
    src = ei[0]
    dst = ei[1]
    zeros = jnp.zeros((N_TILE, 128), jnp.float32)
    batch3d = batch_l.reshape(N // BLK, 1, BLK)

    agg1 = _sc_agg(x_l, src, dst, zeros, False).reshape(2, N, 128)
    r1 = _tc_root(x_l, W1_root, b1, 128)
    h1 = _tc_layer1(agg1, r1, W1_rel)

    agg2 = _sc_agg(h1, src, dst, zeros, False).reshape(2, N, 128)
    r2 = _tc_root(h1, W2_root, b2, 256)
    h2 = _tc_layer2(agg2, r2, W2_rel)

    agg3 = _sc_agg(h2.reshape(2 * N, 128), src, dst, zeros,
                   True).reshape(2, N, 128)
    r3 = _tc_root(h2, W3_root, b3, 256)
    x_atom, x_pool = _tc_layer3_fc_pool(agg3, r3, W3_rel, Wfc,
                                        bfc, gamma, beta, batch3d)
    return (x_atom, x_pool)
